# baseline (speedup 1.0000x reference)
import jax
import jax.numpy as jnp
from jax.experimental import pallas as pl
from jax.experimental.pallas import tpu as pltpu

_BLOCK_N = 128
_I_CHUNK = 32


def _fused_kernel(x_ref, qt_ref, w1t_ref, b1t_ref, w2t_ref, b2t_ref,
                  w3t_ref, b3t_ref, o_ref):
    nb, d = x_ref.shape
    n_params = w3t_ref.shape[0]
    ic = _I_CHUNK

    xt = x_ref[...].T
    xpt = jnp.dot(qt_ref[...], xt, preferred_element_type=jnp.float32)

    j_io = jax.lax.broadcasted_iota(jnp.int32, (d, d), 0)
    i_io = jax.lax.broadcasted_iota(jnp.int32, (d, d), 1)
    trit = (j_io < i_io).astype(jnp.float32)

    for c in range(d // ic):
        m = trit[:, c * ic:(c + 1) * ic]
        xmt = (m[:, :, None] * xpt[:, None, :]).reshape(d, ic * nb)
        xmt = xmt.astype(jnp.bfloat16)

        h = jnp.dot(w1t_ref[...], xmt, preferred_element_type=jnp.float32)
        h = jnp.maximum(h + b1t_ref[...], 0.0).astype(jnp.bfloat16)
        h = jnp.dot(w2t_ref[...], h, preferred_element_type=jnp.float32)
        h = jnp.maximum(h + b2t_ref[...], 0.0).astype(jnp.bfloat16)
        h = (jnp.dot(w3t_ref[...], h, preferred_element_type=jnp.float32)
             + b3t_ref[...])

        for il in range(ic):
            i = c * ic + il
            o_ref[:, i * n_params:(i + 1) * n_params] = (
                h[:, il * nb:(il + 1) * nb].T)


@jax.jit
def _forward(x, Q, W1, b1, W2, b2, W3, b3):
    n, d = x.shape
    n_params = W3.shape[1]
    nb = _BLOCK_N

    xb = x.astype(jnp.bfloat16)
    qt = Q.T.astype(jnp.bfloat16)
    w1t = W1.T.astype(jnp.bfloat16)
    w2t = W2.T.astype(jnp.bfloat16)
    w3t = W3.T.astype(jnp.bfloat16)
    b1t = b1.reshape(-1, 1)
    b2t = b2.reshape(-1, 1)
    b3t = b3.reshape(-1, 1)

    const = lambda i: (0, 0)

    out = pl.pallas_call(
        _fused_kernel,
        grid=(n // nb,),
        in_specs=[
            pl.BlockSpec((nb, d), lambda i: (i, 0)),
            pl.BlockSpec(qt.shape, const),
            pl.BlockSpec(w1t.shape, const),
            pl.BlockSpec(b1t.shape, const),
            pl.BlockSpec(w2t.shape, const),
            pl.BlockSpec(b2t.shape, const),
            pl.BlockSpec(w3t.shape, const),
            pl.BlockSpec(b3t.shape, const),
        ],
        out_specs=pl.BlockSpec((nb, d * n_params), lambda i: (i, 0)),
        out_shape=jax.ShapeDtypeStruct((n, d * n_params), jnp.float32),
        compiler_params=pltpu.CompilerParams(
            dimension_semantics=("parallel",)),
    )(xb, qt, w1t, b1t, w2t, b2t, w3t, b3t)

    return out


def kernel(x, Q, W1, b1, W2, b2, W3, b3):
    return _forward(x, Q, W1, b1, W2, b2, W3, b3)


# speedup vs baseline: 1.1564x; 1.1564x over previous
import jax
import jax.numpy as jnp
from jax.experimental import pallas as pl
from jax.experimental.pallas import tpu as pltpu

_BLOCK_N = 128
_I_CHUNK = 32


def _fused_kernel(x_ref, qt_ref, w1t_ref, b1t_ref, w2t_ref, b2t_ref,
                  w3t_ref, b3t_ref, o_ref):
    nb, d = x_ref.shape
    n_params = w3t_ref.shape[0]
    ic = _I_CHUNK

    xt = x_ref[...].T
    xpt = jnp.dot(qt_ref[...], xt, preferred_element_type=jnp.float32)

    for c in range(d // ic):
        j_io = jax.lax.broadcasted_iota(jnp.int32, (d, ic, nb), 0)
        i_io = jax.lax.broadcasted_iota(jnp.int32, (d, ic, nb), 1) + c * ic
        m = (j_io < i_io).astype(jnp.float32)
        xmt = (m * xpt[:, None, :]).reshape(d, ic * nb).astype(jnp.bfloat16)

        h = jnp.dot(w1t_ref[...], xmt, preferred_element_type=jnp.float32)
        h = jnp.maximum(h + b1t_ref[...], 0.0).astype(jnp.bfloat16)
        h = jnp.dot(w2t_ref[...], h, preferred_element_type=jnp.float32)
        h = jnp.maximum(h + b2t_ref[...], 0.0).astype(jnp.bfloat16)
        h = (jnp.dot(w3t_ref[...], h, preferred_element_type=jnp.float32)
             + b3t_ref[...])

        for il in range(ic):
            i = c * ic + il
            o_ref[:, i * n_params:(i + 1) * n_params] = (
                h[:, il * nb:(il + 1) * nb].T)


@jax.jit
def _forward(x, Q, W1, b1, W2, b2, W3, b3):
    n, d = x.shape
    n_params = W3.shape[1]
    nb = _BLOCK_N

    xb = x.astype(jnp.bfloat16)
    qt = Q.T.astype(jnp.bfloat16)
    w1t = W1.T.astype(jnp.bfloat16)
    w2t = W2.T.astype(jnp.bfloat16)
    w3t = W3.T.astype(jnp.bfloat16)
    b1t = b1.reshape(-1, 1)
    b2t = b2.reshape(-1, 1)
    b3t = b3.reshape(-1, 1)

    const = lambda i: (0, 0)

    out = pl.pallas_call(
        _fused_kernel,
        grid=(n // nb,),
        in_specs=[
            pl.BlockSpec((nb, d), lambda i: (i, 0)),
            pl.BlockSpec(qt.shape, const),
            pl.BlockSpec(w1t.shape, const),
            pl.BlockSpec(b1t.shape, const),
            pl.BlockSpec(w2t.shape, const),
            pl.BlockSpec(b2t.shape, const),
            pl.BlockSpec(w3t.shape, const),
            pl.BlockSpec(b3t.shape, const),
        ],
        out_specs=pl.BlockSpec((nb, d * n_params), lambda i: (i, 0)),
        out_shape=jax.ShapeDtypeStruct((n, d * n_params), jnp.float32),
        compiler_params=pltpu.CompilerParams(
            dimension_semantics=("parallel",)),
    )(xb, qt, w1t, b1t, w2t, b2t, w3t, b3t)

    return out


def kernel(x, Q, W1, b1, W2, b2, W3, b3):
    return _forward(x, Q, W1, b1, W2, b2, W3, b3)
